# baseline (speedup 1.0000x reference)
import functools

import jax
import jax.numpy as jnp
from jax import lax
from jax.experimental import pallas as pl
from jax.experimental.pallas import tpu as pltpu
from jax.experimental.pallas import tpu_sc as plsc

EMBED_DIM = 16
NUM_TABLES = 22
BATCH = 4096

_SIZES = (5, 3604778, 1951876, 432750, 14, 19, 2638598, 1962400, 11123,
          397159, 578622, 3, 622, 107, 8009, 30685, 104, 10, 3, 2, 9, 7)
_SC_TABLES = (1, 2, 3, 6, 7, 8, 9, 10, 12, 14, 15)
_OH_TABLES = (0, 4, 5, 11, 13, 16, 17, 18, 19, 20, 21)
_N_SC = len(_SC_TABLES)
_N_OH = len(_OH_TABLES)
_ET_DIM = _N_SC * EMBED_DIM
_OH_DIM = _N_OH * EMBED_DIM

_SC_INFO = plsc.get_sparse_core_info()
_NC = _SC_INFO.num_cores
_NW = _NC * _SC_INFO.num_subcores
_B_PER_W = BATCH // _NW
_G = 16
_NGROUPS = _B_PER_W // _G


def _sc_gather_body(*refs):
    xT = refs[0]
    tTs = refs[1 : 1 + _N_SC]
    out = refs[1 + _N_SC]
    idx_v = refs[2 + _N_SC]
    rows_vT = refs[3 + _N_SC]
    colbuf = refs[4 + _N_SC]
    sems = refs[5 + _N_SC :]

    wid = lax.axis_index("s") * _NC + lax.axis_index("c")
    base = wid * _B_PER_W
    lanes = lax.iota(jnp.int32, 16)

    for p, i in enumerate(_SC_TABLES):
        tT = tTs[p]
        max_col = ((_SIZES[i] - 1) // 128) * 128
        pltpu.sync_copy(xT.at[i, pl.ds(base, _B_PER_W)], idx_v)

        def fire(g, buf):
            off = pl.multiple_of(g * _G, _G)
            jv = idx_v[pl.ds(off, _G)]
            cols = jnp.clip((jv // 128) * 128, 0, max_col)
            copies = []
            for l in range(_G):
                c = pl.multiple_of(cols[l], 128)
                copies.append(
                    pltpu.async_copy(
                        tT.at[:, pl.ds(c, 128)],
                        colbuf.at[buf, l],
                        sems[buf],
                    )
                )
            return jv, copies

        def extract(g, buf, jv, copies):
            for cp in copies:
                cp.wait()
            jm = jv % 128
            off = pl.multiple_of(g * _G, _G)
            for comp in range(16):
                v = plsc.load_gather(
                    colbuf.at[buf],
                    [lanes, jnp.full((16,), comp, jnp.int32), jm],
                )
                rows_vT[comp, pl.ds(off, _G)] = v

        def pair_body(k, _):
            g0 = pl.multiple_of(4 * k, 4)
            jv0, cp0 = fire(g0, 0)
            jv1, cp1 = fire(g0 + 1, 1)
            jv2, cp2 = fire(g0 + 2, 2)
            extract(g0, 0, jv0, cp0)
            jv3, cp3 = fire(g0 + 3, 0)
            extract(g0 + 1, 1, jv1, cp1)
            extract(g0 + 2, 2, jv2, cp2)
            extract(g0 + 3, 0, jv3, cp3)
            return _

        lax.fori_loop(0, _NGROUPS // 4, pair_body, 0)
        pltpu.sync_copy(
            rows_vT, out.at[pl.ds(p * 16, 16), pl.ds(base, _B_PER_W)]
        )


_sc_gather = functools.partial(
    pl.kernel,
    out_type=jax.ShapeDtypeStruct((_ET_DIM, BATCH), jnp.float32),
    mesh=plsc.VectorSubcoreMesh(core_axis_name="c", subcore_axis_name="s"),
    scratch_types=[
        pltpu.VMEM((_B_PER_W,), jnp.int32),
        pltpu.VMEM((16, _B_PER_W), jnp.float32),
        pltpu.VMEM((3, _G, 16, 128), jnp.float32),
    ] + [pltpu.SemaphoreType.DMA] * 3,
    compiler_params=pltpu.CompilerParams(needs_layout_passes=False),
)(_sc_gather_body)


def _dice(x, alpha, eps=1e-4):
    mean = jnp.mean(x, axis=-1, keepdims=True)
    var = jnp.mean(jnp.square(x - mean), axis=-1, keepdims=True)
    normed = (x - mean) * lax.rsqrt(var + eps)
    p = jax.nn.sigmoid(normed)
    return x * (p + (1.0 - p) * alpha)


def _mlp_body(*refs):
    (et_ref, xoh_ref, w1a_ref, w1b_ref, b1_ref, a1_ref, w2_ref, b2_ref,
     a2_ref) = refs[: 9]
    tab_refs = refs[9 : 9 + _N_OH]
    o_ref = refs[9 + _N_OH]

    h = lax.dot_general(
        et_ref[...], w1a_ref[...], (((0,), (1,)), ((), ())),
        preferred_element_type=jnp.float32,
    )
    es = []
    for t, i in enumerate(_OH_TABLES):
        sz = _SIZES[i]
        idxs = xoh_ref[...][:, t]
        oh = (idxs[:, None] == lax.iota(jnp.int32, sz)[None, :]).astype(
            jnp.float32
        )
        es.append(
            jnp.dot(oh, tab_refs[t][...], preferred_element_type=jnp.float32)
        )
    e_oh = jnp.concatenate(es, axis=-1)
    h = h + lax.dot_general(
        e_oh, w1b_ref[...], (((1,), (1,)), ((), ())),
        preferred_element_type=jnp.float32,
    )
    h = _dice(h + b1_ref[...], a1_ref[...])
    o = lax.dot_general(
        h, w2_ref[...], (((1,), (1,)), ((), ())),
        preferred_element_type=jnp.float32,
    ) + b2_ref[...]
    o_ref[...] = _dice(o, a2_ref[...])


def _mlp(et, x_oh, W1A, W1B, b1, alpha1, W2, b2, alpha2, tabs, block_m=1024):
    n_blocks = BATCH // block_m
    grid_spec = pl.GridSpec(
        grid=(n_blocks,),
        in_specs=[
            pl.BlockSpec((_ET_DIM, block_m), lambda i: (0, i)),
            pl.BlockSpec((block_m, _N_OH), lambda i: (i, 0)),
            pl.BlockSpec((1024, _ET_DIM), lambda i: (0, 0)),
            pl.BlockSpec((1024, _OH_DIM), lambda i: (0, 0)),
            pl.BlockSpec((1, 1024), lambda i: (0, 0)),
            pl.BlockSpec((1, 1024), lambda i: (0, 0)),
            pl.BlockSpec((512, 1024), lambda i: (0, 0)),
            pl.BlockSpec((1, 512), lambda i: (0, 0)),
            pl.BlockSpec((1, 512), lambda i: (0, 0)),
        ] + [
            pl.BlockSpec((_SIZES[i], EMBED_DIM), lambda i_: (0, 0))
            for i in _OH_TABLES
        ],
        out_specs=pl.BlockSpec((block_m, 512), lambda i: (i, 0)),
    )
    return pl.pallas_call(
        _mlp_body,
        grid_spec=grid_spec,
        out_shape=jax.ShapeDtypeStruct((BATCH, 512), jnp.float32),
    )(et, x_oh, W1A, W1B, b1.reshape(1, -1), alpha1, W2, b2.reshape(1, -1),
      alpha2, *tabs)


@jax.jit
def kernel(x, tables, W1, b1, W2, b2, alpha1, alpha2):
    xT = x.T
    tTs = [tables[i].T for i in _SC_TABLES]
    et = _sc_gather(xT, *tTs)
    W1A = jnp.concatenate(
        [W1[:, 16 * i : 16 * i + 16] for i in _SC_TABLES], axis=1
    )
    W1B = jnp.concatenate(
        [W1[:, 16 * i : 16 * i + 16] for i in _OH_TABLES], axis=1
    )
    x_oh = jnp.stack([x[:, i] for i in _OH_TABLES], axis=1)
    tabs = [tables[i] for i in _OH_TABLES]
    return _mlp(et, x_oh, W1A, W1B, b1, alpha1, W2, b2, alpha2, tabs)


# speedup vs baseline: 1.0057x; 1.0057x over previous
import functools

import jax
import jax.numpy as jnp
from jax import lax
from jax.experimental import pallas as pl
from jax.experimental.pallas import tpu as pltpu
from jax.experimental.pallas import tpu_sc as plsc

EMBED_DIM = 16
NUM_TABLES = 22
BATCH = 4096

_SIZES = (5, 3604778, 1951876, 432750, 14, 19, 2638598, 1962400, 11123,
          397159, 578622, 3, 622, 107, 8009, 30685, 104, 10, 3, 2, 9, 7)
_SC_TABLES = (1, 2, 3, 6, 7, 8, 9, 10, 12, 14, 15)
_OH_TABLES = (0, 4, 5, 11, 13, 16, 17, 18, 19, 20, 21)
_N_SC = len(_SC_TABLES)
_N_OH = len(_OH_TABLES)
_ET_DIM = _N_SC * EMBED_DIM
_OH_DIM = _N_OH * EMBED_DIM

_SC_INFO = plsc.get_sparse_core_info()
_NC = _SC_INFO.num_cores
_NW = _NC * _SC_INFO.num_subcores
_B_PER_W = BATCH // _NW
_G = 16
_NGROUPS = _B_PER_W // _G


def _sc_gather_body(*refs):
    xT = refs[0]
    tTs = refs[1 : 1 + _N_SC]
    out = refs[1 + _N_SC]
    idx_v = refs[2 + _N_SC]
    rows_vT = refs[3 + _N_SC]
    colbuf = refs[4 + _N_SC]
    sems = refs[5 + _N_SC :]

    wid = lax.axis_index("s") * _NC + lax.axis_index("c")
    base = wid * _B_PER_W
    lanes = lax.iota(jnp.int32, 16)

    for p, i in enumerate(_SC_TABLES):
        tT = tTs[p]
        max_col = ((_SIZES[i] - 1) // 128) * 128
        pltpu.sync_copy(xT.at[i, pl.ds(base, _B_PER_W)], idx_v)

        def fire(g, buf):
            off = pl.multiple_of(g * _G, _G)
            jv = idx_v[pl.ds(off, _G)]
            cols = jnp.clip((jv // 128) * 128, 0, max_col)
            copies = []
            for l in range(_G):
                c = pl.multiple_of(cols[l], 128)
                copies.append(
                    pltpu.async_copy(
                        tT.at[:, pl.ds(c, 128)],
                        colbuf.at[buf, l],
                        sems[buf],
                    )
                )
            return jv, copies

        def extract(g, buf, jv, copies):
            for cp in copies:
                cp.wait()
            jm = jv % 128
            off = pl.multiple_of(g * _G, _G)
            for comp in range(16):
                v = plsc.load_gather(
                    colbuf.at[buf],
                    [lanes, jnp.full((16,), comp, jnp.int32), jm],
                )
                rows_vT[comp, pl.ds(off, _G)] = v

        def pair_body(k, _):
            g0 = pl.multiple_of(2 * k, 2)
            jv0, cp0 = fire(g0, 0)
            jv1, cp1 = fire(g0 + 1, 1)
            extract(g0, 0, jv0, cp0)
            extract(g0 + 1, 1, jv1, cp1)
            return _

        lax.fori_loop(0, _NGROUPS // 2, pair_body, 0)
        pltpu.sync_copy(
            rows_vT, out.at[pl.ds(p * 16, 16), pl.ds(base, _B_PER_W)]
        )


_sc_gather = functools.partial(
    pl.kernel,
    out_type=jax.ShapeDtypeStruct((_ET_DIM, BATCH), jnp.float32),
    mesh=plsc.VectorSubcoreMesh(core_axis_name="c", subcore_axis_name="s"),
    scratch_types=[
        pltpu.VMEM((_B_PER_W,), jnp.int32),
        pltpu.VMEM((16, _B_PER_W), jnp.float32),
        pltpu.VMEM((3, _G, 16, 128), jnp.float32),
    ] + [pltpu.SemaphoreType.DMA] * 3,
    compiler_params=pltpu.CompilerParams(needs_layout_passes=False),
)(_sc_gather_body)


def _dice(x, alpha, eps=1e-4):
    mean = jnp.mean(x, axis=-1, keepdims=True)
    var = jnp.mean(jnp.square(x - mean), axis=-1, keepdims=True)
    normed = (x - mean) * lax.rsqrt(var + eps)
    p = jax.nn.sigmoid(normed)
    return x * (p + (1.0 - p) * alpha)


def _mlp_body(*refs):
    (et_ref, xoh_ref, w1a_ref, w1b_ref, b1_ref, a1_ref, w2_ref, b2_ref,
     a2_ref) = refs[: 9]
    tab_refs = refs[9 : 9 + _N_OH]
    o_ref = refs[9 + _N_OH]

    bf = jnp.bfloat16
    h = lax.dot_general(
        et_ref[...].astype(bf), w1a_ref[...].astype(bf),
        (((0,), (1,)), ((), ())),
        preferred_element_type=jnp.float32,
    )
    es = []
    for t, i in enumerate(_OH_TABLES):
        sz = _SIZES[i]
        idxs = xoh_ref[...][:, t]
        oh = (idxs[:, None] == lax.iota(jnp.int32, sz)[None, :]).astype(bf)
        es.append(
            jnp.dot(oh, tab_refs[t][...].astype(bf),
                    preferred_element_type=jnp.float32)
        )
    e_oh = jnp.concatenate(es, axis=-1)
    h = h + lax.dot_general(
        e_oh.astype(bf), w1b_ref[...].astype(bf), (((1,), (1,)), ((), ())),
        preferred_element_type=jnp.float32,
    )
    h = _dice(h + b1_ref[...], a1_ref[...])
    o = lax.dot_general(
        h.astype(bf), w2_ref[...].astype(bf), (((1,), (1,)), ((), ())),
        preferred_element_type=jnp.float32,
    ) + b2_ref[...]
    o_ref[...] = _dice(o, a2_ref[...])


def _mlp(et, x_oh, W1A, W1B, b1, alpha1, W2, b2, alpha2, tabs, block_m=1024):
    n_blocks = BATCH // block_m
    grid_spec = pl.GridSpec(
        grid=(n_blocks,),
        in_specs=[
            pl.BlockSpec((_ET_DIM, block_m), lambda i: (0, i)),
            pl.BlockSpec((block_m, _N_OH), lambda i: (i, 0)),
            pl.BlockSpec((1024, _ET_DIM), lambda i: (0, 0)),
            pl.BlockSpec((1024, _OH_DIM), lambda i: (0, 0)),
            pl.BlockSpec((1, 1024), lambda i: (0, 0)),
            pl.BlockSpec((1, 1024), lambda i: (0, 0)),
            pl.BlockSpec((512, 1024), lambda i: (0, 0)),
            pl.BlockSpec((1, 512), lambda i: (0, 0)),
            pl.BlockSpec((1, 512), lambda i: (0, 0)),
        ] + [
            pl.BlockSpec((_SIZES[i], EMBED_DIM), lambda i_: (0, 0))
            for i in _OH_TABLES
        ],
        out_specs=pl.BlockSpec((block_m, 512), lambda i: (i, 0)),
    )
    return pl.pallas_call(
        _mlp_body,
        grid_spec=grid_spec,
        out_shape=jax.ShapeDtypeStruct((BATCH, 512), jnp.float32),
    )(et, x_oh, W1A, W1B, b1.reshape(1, -1), alpha1, W2, b2.reshape(1, -1),
      alpha2, *tabs)


@jax.jit
def kernel(x, tables, W1, b1, W2, b2, alpha1, alpha2):
    xT = x.T
    tTs = [tables[i].T for i in _SC_TABLES]
    et = _sc_gather(xT, *tTs)
    W1A = jnp.concatenate(
        [W1[:, 16 * i : 16 * i + 16] for i in _SC_TABLES], axis=1
    )
    W1B = jnp.concatenate(
        [W1[:, 16 * i : 16 * i + 16] for i in _OH_TABLES], axis=1
    )
    x_oh = jnp.stack([x[:, i] for i in _OH_TABLES], axis=1)
    tabs = [tables[i] for i in _OH_TABLES]
    return _mlp(et, x_oh, W1A, W1B, b1, alpha1, W2, b2, alpha2, tabs)


# speedup vs baseline: 1.2468x; 1.2397x over previous
import functools

import jax
import jax.numpy as jnp
from jax import lax
from jax.experimental import pallas as pl
from jax.experimental.pallas import tpu as pltpu
from jax.experimental.pallas import tpu_sc as plsc

EMBED_DIM = 16
NUM_TABLES = 22
BATCH = 4096

_SIZES = (5, 3604778, 1951876, 432750, 14, 19, 2638598, 1962400, 11123,
          397159, 578622, 3, 622, 107, 8009, 30685, 104, 10, 3, 2, 9, 7)
_SC_TABLES = (1, 2, 3, 6, 7, 8, 9, 10, 12, 14, 15)
_OH_TABLES = (0, 4, 5, 11, 13, 16, 17, 18, 19, 20, 21)
_N_SC = len(_SC_TABLES)
_N_OH = len(_OH_TABLES)
_ET_DIM = _N_SC * EMBED_DIM
_OH_DIM = _N_OH * EMBED_DIM

_SC_INFO = plsc.get_sparse_core_info()
_NC = _SC_INFO.num_cores
_NW = _NC * _SC_INFO.num_subcores
_B_PER_W = BATCH // _NW
_G = 16
_NGROUPS = _B_PER_W // _G


def _sc_gather_body(*refs):
    xT = refs[0]
    tTs = refs[1 : 1 + _N_SC]
    out = refs[1 + _N_SC]
    idx_v = refs[2 + _N_SC]
    rows_vT = refs[3 + _N_SC]
    colbuf = refs[4 + _N_SC]
    sems = refs[5 + _N_SC :]

    wid = lax.axis_index("s") * _NC + lax.axis_index("c")
    base = wid * _B_PER_W
    lanes = lax.iota(jnp.int32, 16)

    for p, i in enumerate(_SC_TABLES):
        tT = tTs[p]
        max_col = ((_SIZES[i] - 1) // 128) * 128
        tn = (_SIZES[i] + 127) // 128
        pltpu.sync_copy(xT.at[i, pl.ds(base, _B_PER_W)], idx_v)

        if tn <= _G:
            copies = [
                pltpu.async_copy(
                    tT.at[
                        :,
                        pl.ds(pl.multiple_of(jnp.int32(t) * 128, 128), 128),
                    ],
                    colbuf.at[0, t],
                    sems[0],
                )
                for t in range(tn)
            ]
            for cp in copies:
                cp.wait()
            for g in range(_NGROUPS):
                off = g * _G
                jv = idx_v[pl.ds(off, _G)]
                jdiv = jnp.clip(jv // 128, 0, tn - 1)
                jm = jv % 128
                for comp in range(16):
                    v = plsc.load_gather(
                        colbuf.at[0],
                        [jdiv, jnp.full((16,), comp, jnp.int32), jm],
                    )
                    rows_vT[comp, pl.ds(off, _G)] = v
            pltpu.sync_copy(
                rows_vT, out.at[pl.ds(p * 16, 16), pl.ds(base, _B_PER_W)]
            )
            continue

        def fire(g, buf):
            off = pl.multiple_of(g * _G, _G)
            jv = idx_v[pl.ds(off, _G)]
            cols = jnp.clip((jv // 128) * 128, 0, max_col)
            copies = []
            for l in range(_G):
                c = pl.multiple_of(cols[l], 128)
                copies.append(
                    pltpu.async_copy(
                        tT.at[:, pl.ds(c, 128)],
                        colbuf.at[buf, l],
                        sems[buf],
                    )
                )
            return jv, copies

        def extract(g, buf, jv, copies):
            for cp in copies:
                cp.wait()
            jm = jv % 128
            off = pl.multiple_of(g * _G, _G)
            for comp in range(16):
                v = plsc.load_gather(
                    colbuf.at[buf],
                    [lanes, jnp.full((16,), comp, jnp.int32), jm],
                )
                rows_vT[comp, pl.ds(off, _G)] = v

        def pair_body(k, _):
            g0 = pl.multiple_of(2 * k, 2)
            jv0, cp0 = fire(g0, 0)
            jv1, cp1 = fire(g0 + 1, 1)
            extract(g0, 0, jv0, cp0)
            extract(g0 + 1, 1, jv1, cp1)
            return _

        lax.fori_loop(0, _NGROUPS // 2, pair_body, 0)
        pltpu.sync_copy(
            rows_vT, out.at[pl.ds(p * 16, 16), pl.ds(base, _B_PER_W)]
        )


_sc_gather = functools.partial(
    pl.kernel,
    out_type=jax.ShapeDtypeStruct((_ET_DIM, BATCH), jnp.float32),
    mesh=plsc.VectorSubcoreMesh(core_axis_name="c", subcore_axis_name="s"),
    scratch_types=[
        pltpu.VMEM((_B_PER_W,), jnp.int32),
        pltpu.VMEM((16, _B_PER_W), jnp.float32),
        pltpu.VMEM((3, _G, 16, 128), jnp.float32),
    ] + [pltpu.SemaphoreType.DMA] * 3,
    compiler_params=pltpu.CompilerParams(needs_layout_passes=False),
)(_sc_gather_body)


def _dice(x, alpha, eps=1e-4):
    mean = jnp.mean(x, axis=-1, keepdims=True)
    var = jnp.mean(jnp.square(x - mean), axis=-1, keepdims=True)
    normed = (x - mean) * lax.rsqrt(var + eps)
    p = jax.nn.sigmoid(normed)
    return x * (p + (1.0 - p) * alpha)


def _mlp_body(*refs):
    (et_ref, xoh_ref, w1a_ref, w1b_ref, b1_ref, a1_ref, w2_ref, b2_ref,
     a2_ref) = refs[: 9]
    tab_refs = refs[9 : 9 + _N_OH]
    o_ref = refs[9 + _N_OH]

    bf = jnp.bfloat16
    h = lax.dot_general(
        et_ref[...].astype(bf), w1a_ref[...].astype(bf),
        (((0,), (1,)), ((), ())),
        preferred_element_type=jnp.float32,
    )
    es = []
    for t, i in enumerate(_OH_TABLES):
        sz = _SIZES[i]
        idxs = xoh_ref[...][:, t]
        oh = (idxs[:, None] == lax.iota(jnp.int32, sz)[None, :]).astype(bf)
        es.append(
            jnp.dot(oh, tab_refs[t][...].astype(bf),
                    preferred_element_type=jnp.float32)
        )
    e_oh = jnp.concatenate(es, axis=-1)
    h = h + lax.dot_general(
        e_oh.astype(bf), w1b_ref[...].astype(bf), (((1,), (1,)), ((), ())),
        preferred_element_type=jnp.float32,
    )
    h = _dice(h + b1_ref[...], a1_ref[...])
    o = lax.dot_general(
        h.astype(bf), w2_ref[...].astype(bf), (((1,), (1,)), ((), ())),
        preferred_element_type=jnp.float32,
    ) + b2_ref[...]
    o_ref[...] = _dice(o, a2_ref[...])


def _mlp(et, x_oh, W1A, W1B, b1, alpha1, W2, b2, alpha2, tabs, block_m=1024):
    n_blocks = BATCH // block_m
    grid_spec = pl.GridSpec(
        grid=(n_blocks,),
        in_specs=[
            pl.BlockSpec((_ET_DIM, block_m), lambda i: (0, i)),
            pl.BlockSpec((block_m, _N_OH), lambda i: (i, 0)),
            pl.BlockSpec((1024, _ET_DIM), lambda i: (0, 0)),
            pl.BlockSpec((1024, _OH_DIM), lambda i: (0, 0)),
            pl.BlockSpec((1, 1024), lambda i: (0, 0)),
            pl.BlockSpec((1, 1024), lambda i: (0, 0)),
            pl.BlockSpec((512, 1024), lambda i: (0, 0)),
            pl.BlockSpec((1, 512), lambda i: (0, 0)),
            pl.BlockSpec((1, 512), lambda i: (0, 0)),
        ] + [
            pl.BlockSpec((_SIZES[i], EMBED_DIM), lambda i_: (0, 0))
            for i in _OH_TABLES
        ],
        out_specs=pl.BlockSpec((block_m, 512), lambda i: (i, 0)),
    )
    return pl.pallas_call(
        _mlp_body,
        grid_spec=grid_spec,
        out_shape=jax.ShapeDtypeStruct((BATCH, 512), jnp.float32),
    )(et, x_oh, W1A, W1B, b1.reshape(1, -1), alpha1, W2, b2.reshape(1, -1),
      alpha2, *tabs)


@jax.jit
def kernel(x, tables, W1, b1, W2, b2, alpha1, alpha2):
    xT = x.T
    tTs = [tables[i].T for i in _SC_TABLES]
    et = _sc_gather(xT, *tTs)
    W1A = jnp.concatenate(
        [W1[:, 16 * i : 16 * i + 16] for i in _SC_TABLES], axis=1
    )
    W1B = jnp.concatenate(
        [W1[:, 16 * i : 16 * i + 16] for i in _OH_TABLES], axis=1
    )
    x_oh = jnp.stack([x[:, i] for i in _OH_TABLES], axis=1)
    tabs = [tables[i] for i in _OH_TABLES]
    return _mlp(et, x_oh, W1A, W1B, b1, alpha1, W2, b2, alpha2, tabs)
